# Initial kernel scaffold; baseline (speedup 1.0000x reference)
import functools

import jax
import jax.numpy as jnp
from jax import lax
from jax.experimental import pallas as pl
from jax.experimental.pallas import tpu as pltpu
from jax.experimental.pallas import tpu_sc as plsc

N = 10000
E = 160000
NP = 10240
EP = 163840
H = 32
DIN = 128
NW = 32
CH = 128
NCHUNK = EP // (NW * CH)
PERW = EP // NW
ROWS_PER_TILE = NP // 16

_mesh = plsc.VectorSubcoreMesh(core_axis_name="c", subcore_axis_name="s")


def _worker_id():
    return lax.axis_index("s") * 2 + lax.axis_index("c")


@functools.partial(
    pl.kernel,
    out_type=jax.ShapeDtypeStruct((2, NP), jnp.float32),
    mesh=_mesh,
    scratch_types=[
        pltpu.VMEM((CH,), jnp.int32),
        pltpu.VMEM((CH,), jnp.float32),
        pltpu.VMEM((ROWS_PER_TILE,), jnp.float32),
        pltpu.VMEM_SHARED((NP,), jnp.float32),
    ],
)
def _sc_degree(dst_hbm, degp_hbm, idx_v, ones_v, zero_v, deg_sp):
    c = lax.axis_index("c")
    s = lax.axis_index("s")
    w = _worker_id()
    z16 = jnp.zeros((16,), jnp.float32)
    o16 = jnp.ones((16,), jnp.float32)

    def fill_zero(i, carry):
        zero_v[pl.ds(i * 16, 16)] = z16
        return carry

    lax.fori_loop(0, ROWS_PER_TILE // 16, fill_zero, 0)

    def fill_one(i, carry):
        ones_v[pl.ds(i * 16, 16)] = o16
        return carry

    lax.fori_loop(0, CH // 16, fill_one, 0)
    pltpu.sync_copy(zero_v, deg_sp.at[pl.ds(s * ROWS_PER_TILE, ROWS_PER_TILE)])
    plsc.subcore_barrier()

    def body(j, carry):
        base = pl.multiple_of(w * PERW + j * CH, CH)
        pltpu.sync_copy(dst_hbm.at[pl.ds(base, CH)], idx_v)
        pltpu.sync_copy(ones_v, deg_sp.at[idx_v], add=True)
        return carry

    lax.fori_loop(0, NCHUNK, body, 0)
    plsc.subcore_barrier()
    pltpu.sync_copy(deg_sp.at[pl.ds(s * ROWS_PER_TILE, ROWS_PER_TILE)],
                    degp_hbm.at[c, pl.ds(s * ROWS_PER_TILE, ROWS_PER_TILE)])


@functools.partial(
    pl.kernel,
    out_type=jax.ShapeDtypeStruct((EP,), jnp.float32),
    mesh=_mesh,
    scratch_types=[
        pltpu.VMEM((CH,), jnp.int32),
        pltpu.VMEM((CH,), jnp.int32),
        pltpu.VMEM((CH,), jnp.float32),
        pltpu.VMEM((NP, 2), jnp.float32),
        pltpu.VMEM((NP,), jnp.float32),
    ],
)
def _sc_gate(dst_hbm, src_hbm, ab_hbm, deg_hbm, e_hbm,
             di_v, si_v, ev_v, ab_v, deg_v):
    w = _worker_id()
    pltpu.sync_copy(ab_hbm, ab_v)
    pltpu.sync_copy(deg_hbm, deg_v)
    zi16 = jnp.zeros((16,), jnp.int32)
    oi16 = jnp.ones((16,), jnp.int32)

    def body(j, carry):
        base = pl.multiple_of(w * PERW + j * CH, CH)
        pltpu.sync_copy(dst_hbm.at[pl.ds(base, CH)], di_v)
        pltpu.sync_copy(src_hbm.at[pl.ds(base, CH)], si_v)
        for r in range(CH // 16):
            d16 = di_v[pl.ds(r * 16, 16)]
            s16 = si_v[pl.ds(r * 16, 16)]
            av = plsc.load_gather(ab_v, [d16, zi16])
            bv = plsc.load_gather(ab_v, [s16, oi16])
            dd = plsc.load_gather(deg_v, [d16])
            ds_ = plsc.load_gather(deg_v, [s16])
            zz = av + bv
            t = 1.0 - 2.0 / (jnp.exp(2.0 * zz) + 1.0)
            ev_v[pl.ds(r * 16, 16)] = 0.3 + t * dd * ds_
        pltpu.sync_copy(ev_v, e_hbm.at[pl.ds(base, CH)])
        return carry

    lax.fori_loop(0, NCHUNK, body, 0)


@functools.partial(
    pl.kernel,
    out_type=jax.ShapeDtypeStruct((EP, H), jnp.float32),
    mesh=_mesh,
    scratch_types=[
        pltpu.VMEM((CH,), jnp.int32),
        pltpu.VMEM((CH, H), jnp.float32),
        pltpu.SemaphoreType.DMA,
    ],
)
def _sc_gather(table_hbm, src_hbm, out_hbm, idx_v, rows_v, sem):
    w = _worker_id()

    def body(j, carry):
        base = pl.multiple_of(w * PERW + j * CH, CH)
        pltpu.sync_copy(src_hbm.at[pl.ds(base, CH)], idx_v)
        pltpu.async_copy(table_hbm.at[idx_v], rows_v, sem).wait()
        pltpu.sync_copy(rows_v, out_hbm.at[pl.ds(base, CH), :])
        return carry

    lax.fori_loop(0, NCHUNK, body, 0)


@functools.partial(
    pl.kernel,
    out_type=jax.ShapeDtypeStruct((2, NP, H), jnp.float32),
    mesh=_mesh,
    scratch_types=[
        pltpu.VMEM((CH,), jnp.int32),
        pltpu.VMEM((CH, H), jnp.float32),
        pltpu.VMEM((CH, H), jnp.float32),
        pltpu.VMEM_SHARED((NP, H), jnp.float32),
    ],
)
def _sc_scatter(msg_hbm, dst_hbm, aggp_hbm, idx_v, msg_v, zero_v, agg_sp):
    c = lax.axis_index("c")
    s = lax.axis_index("s")
    w = _worker_id()
    z16 = jnp.zeros((16,), jnp.float32)

    def fill_zero(i, carry):
        zero_v[i, pl.ds(0, 16)] = z16
        zero_v[i, pl.ds(16, 16)] = z16
        return carry

    lax.fori_loop(0, CH, fill_zero, 0)
    for q in range(ROWS_PER_TILE // CH):
        pltpu.sync_copy(zero_v,
                        agg_sp.at[pl.ds(s * ROWS_PER_TILE + q * CH, CH), :])
    plsc.subcore_barrier()

    def body(j, carry):
        base = pl.multiple_of(w * PERW + j * CH, CH)
        pltpu.sync_copy(dst_hbm.at[pl.ds(base, CH)], idx_v)
        pltpu.sync_copy(msg_hbm.at[pl.ds(base, CH), :], msg_v)
        pltpu.sync_copy(msg_v, agg_sp.at[idx_v], add=True)
        return carry

    lax.fori_loop(0, NCHUNK, body, 0)
    plsc.subcore_barrier()
    pltpu.sync_copy(agg_sp.at[pl.ds(s * ROWS_PER_TILE, ROWS_PER_TILE), :],
                    aggp_hbm.at[c, pl.ds(s * ROWS_PER_TILE, ROWS_PER_TILE), :])


def _tc_project_body(x_ref, wpt_ref, bp_ref, gab_ref, gbh_ref,
                     h_ref, ab_ref):
    hv = jnp.maximum(
        jnp.dot(x_ref[...], wpt_ref[...],
                preferred_element_type=jnp.float32) + bp_ref[...], 0.0)
    h_ref[...] = hv
    ab_ref[...] = jnp.dot(hv, gab_ref[...],
                          preferred_element_type=jnp.float32) + gbh_ref[...]


def _tc_project(xp, wpt, bp_row, gab, gbh):
    return pl.pallas_call(
        _tc_project_body,
        out_shape=(jax.ShapeDtypeStruct((NP, H), jnp.float32),
                   jax.ShapeDtypeStruct((NP, 2), jnp.float32)),
    )(xp, wpt, bp_row, gab, gbh)


_BE = 2048


def _tc_msg_body(g_ref, ea_ref, e_ref, w2_ref, msg_ref):
    u = jnp.dot(g_ref[...], w2_ref[...], preferred_element_type=jnp.float32)
    acc = u[:, 544:576] + e_ref[...] * u[:, 512:544]
    for k in range(16):
        acc = acc + ea_ref[:, k:k + 1] * u[:, 32 * k:32 * k + 32]
    msg_ref[...] = acc


def _tc_msg(g, ea, ev, w2):
    return pl.pallas_call(
        _tc_msg_body,
        grid=(EP // _BE,),
        in_specs=[
            pl.BlockSpec((_BE, H), lambda i: (i, 0)),
            pl.BlockSpec((_BE, 16), lambda i: (i, 0)),
            pl.BlockSpec((_BE, 1), lambda i: (i, 0)),
            pl.BlockSpec((H, 576), lambda i: (0, 0)),
        ],
        out_specs=pl.BlockSpec((_BE, H), lambda i: (i, 0)),
        out_shape=jax.ShapeDtypeStruct((EP, H), jnp.float32),
    )(g, ea, ev, w2)


def _tc_gru_body(aggp_ref, hid_ref, bconv_ref, wih_ref, whh_ref,
                 bih_ref, bhh_ref, out_ref):
    agg = aggp_ref[0] + aggp_ref[1] + bconv_ref[...]
    xg = jnp.maximum(agg, 0.0)
    hid = hid_ref[...]
    gi = jnp.dot(xg, wih_ref[...],
                 preferred_element_type=jnp.float32) + bih_ref[...]
    gh = jnp.dot(hid, whh_ref[...],
                 preferred_element_type=jnp.float32) + bhh_ref[...]
    r = jax.nn.sigmoid(gi[:, :H] + gh[:, :H])
    z = jax.nn.sigmoid(gi[:, H:2 * H] + gh[:, H:2 * H])
    n = jnp.tanh(gi[:, 2 * H:] + r * gh[:, 2 * H:])
    out_ref[...] = (1.0 - z) * n + z * hid


def _tc_gru(aggp, hidden, bconv_row, wih_t, whh_t, bih_row, bhh_row):
    return pl.pallas_call(
        _tc_gru_body,
        out_shape=jax.ShapeDtypeStruct((NP, H), jnp.float32),
    )(aggp, hidden, bconv_row, wih_t, whh_t, bih_row, bhh_row)


def _tc_readout_body(h_ref, nf_ref, wih_ref, whh_ref, blstm_ref,
                     wst_ref, bs_ref, pa_ref, out_ref):
    na = jnp.concatenate([h_ref[:N, :], nf_ref[:N, :]], axis=1)
    hl = jnp.zeros((1, 64), jnp.float32)
    cl = jnp.zeros((1, 64), jnp.float32)
    qs = jnp.zeros((1, 128), jnp.float32)
    for _ in range(3):
        gates = (jnp.dot(qs, wih_ref[...], preferred_element_type=jnp.float32)
                 + jnp.dot(hl, whh_ref[...],
                           preferred_element_type=jnp.float32)
                 + blstm_ref[...])
        i = jax.nn.sigmoid(gates[:, :64])
        f = jax.nn.sigmoid(gates[:, 64:128])
        g = jnp.tanh(gates[:, 128:192])
        o = jax.nn.sigmoid(gates[:, 192:])
        cl = f * cl + i * g
        hl = o * jnp.tanh(cl)
        esc = jnp.sum(na * hl, axis=1, keepdims=True)
        m = jnp.max(esc)
        al = jnp.exp(esc - m)
        al = al / jnp.sum(al)
        ro = jnp.sum(al * na, axis=0, keepdims=True)
        qs = jnp.concatenate([hl, ro], axis=1)
    out = jnp.dot(qs, wst_ref[...],
                  preferred_element_type=jnp.float32) + bs_ref[...]
    pa = pa_ref[0, 0]
    out_ref[...] = jnp.where(out >= 0.0, out, pa * out)


def _tc_readout(h, nf, lwih_t, lwhh_t, blstm_row, ws_t, bs_row, pa):
    return pl.pallas_call(
        _tc_readout_body,
        out_shape=jax.ShapeDtypeStruct((1, 1024), jnp.float32),
    )(h, nf, lwih_t, lwhh_t, blstm_row, ws_t, bs_row, pa)


def kernel(x, edge_index, edge_attr, Wp, bp, gW, gb, Wen, ben, b_conv,
           gru_Wih, gru_Whh, gru_bih, gru_bhh, lstm_Wih, lstm_Whh,
           lstm_bih, lstm_bhh, Ws, bs, prelu_a):
    f32 = jnp.float32
    src = edge_index[0]
    dst = edge_index[1]
    pad = jnp.full((EP - E,), N, jnp.int32)
    src_p = jnp.concatenate([src, pad])
    dst_p = jnp.concatenate([dst, pad])
    ea_p = jnp.zeros((EP, 16), f32).at[:E].set(edge_attr)
    xp = jnp.zeros((NP, DIN), f32).at[:N].set(x)

    wpt = Wp.T
    bp_row = bp[None, :]
    gab = jnp.stack([gW[0, :H], gW[0, H:]], axis=1)
    gbh = jnp.full((1, 2), gb[0] * 0.5, f32)
    w2 = jnp.concatenate(
        [Wen.reshape(H, H, 17).transpose(0, 2, 1).reshape(H, 544),
         ben.reshape(H, H)], axis=1)
    wih_t = gru_Wih.T
    whh_t = gru_Whh.T
    bih_row = gru_bih[None, :]
    bhh_row = gru_bhh[None, :]
    bconv_row = b_conv[None, :]
    lwih_t = lstm_Wih.T
    lwhh_t = lstm_Whh.T
    blstm_row = (lstm_bih + lstm_bhh)[None, :]
    ws_t = Ws.T
    bs_row = bs[None, :]
    pa = jnp.reshape(prelu_a, (1, 1)).astype(f32)

    h, ab = _tc_project(xp, wpt, bp_row, gab, gbh)
    degp = _sc_degree(dst_p)
    deg = degp[0] + degp[1]
    ev = _sc_gate(dst_p, src_p, ab, deg)
    ev2 = ev[:, None]

    hidden = h
    nf = h
    for _ in range(3):
        g = _sc_gather(nf, src_p)
        msg = _tc_msg(g, ea_p, ev2, w2)
        aggp = _sc_scatter(msg, dst_p)
        hidden = _tc_gru(aggp, hidden, bconv_row, wih_t, whh_t,
                         bih_row, bhh_row)
        nf = hidden

    return _tc_readout(h, nf, lwih_t, lwhh_t, blstm_row, ws_t, bs_row, pa)


# speedup vs baseline: 2.0009x; 2.0009x over previous
import functools

import jax
import jax.numpy as jnp
from jax import lax
from jax.experimental import pallas as pl
from jax.experimental.pallas import tpu as pltpu
from jax.experimental.pallas import tpu_sc as plsc

N = 10000
E = 160000
NP = 10240
EP = 163840
H = 32
DIN = 128
NW = 32
CH = 128
NCHUNK = EP // (NW * CH)
PERW = EP // NW
ROWS_PER_TILE = NP // 16

_mesh = plsc.VectorSubcoreMesh(core_axis_name="c", subcore_axis_name="s")


def _worker_id():
    return lax.axis_index("s") * 2 + lax.axis_index("c")


@functools.partial(
    pl.kernel,
    out_type=jax.ShapeDtypeStruct((2, NP), jnp.float32),
    mesh=_mesh,
    scratch_types=[
        pltpu.VMEM((CH,), jnp.int32),
        pltpu.VMEM((CH,), jnp.float32),
        pltpu.VMEM((ROWS_PER_TILE,), jnp.float32),
        pltpu.VMEM_SHARED((NP,), jnp.float32),
    ],
)
def _sc_degree(dst_hbm, degp_hbm, idx_v, ones_v, zero_v, deg_sp):
    c = lax.axis_index("c")
    s = lax.axis_index("s")
    w = _worker_id()
    z16 = jnp.zeros((16,), jnp.float32)
    o16 = jnp.ones((16,), jnp.float32)

    def fill_zero(i, carry):
        zero_v[pl.ds(i * 16, 16)] = z16
        return carry

    lax.fori_loop(0, ROWS_PER_TILE // 16, fill_zero, 0)

    def fill_one(i, carry):
        ones_v[pl.ds(i * 16, 16)] = o16
        return carry

    lax.fori_loop(0, CH // 16, fill_one, 0)
    pltpu.sync_copy(zero_v, deg_sp.at[pl.ds(s * ROWS_PER_TILE, ROWS_PER_TILE)])
    plsc.subcore_barrier()

    def body(j, carry):
        base = pl.multiple_of(w * PERW + j * CH, CH)
        pltpu.sync_copy(dst_hbm.at[pl.ds(base, CH)], idx_v)
        pltpu.sync_copy(ones_v, deg_sp.at[idx_v], add=True)
        return carry

    lax.fori_loop(0, NCHUNK, body, 0)
    plsc.subcore_barrier()
    pltpu.sync_copy(deg_sp.at[pl.ds(s * ROWS_PER_TILE, ROWS_PER_TILE)],
                    degp_hbm.at[c, pl.ds(s * ROWS_PER_TILE, ROWS_PER_TILE)])


@functools.partial(
    pl.kernel,
    out_type=jax.ShapeDtypeStruct((EP,), jnp.float32),
    mesh=_mesh,
    compiler_params=pltpu.CompilerParams(needs_layout_passes=False),
    scratch_types=[
        pltpu.VMEM((CH,), jnp.int32),
        pltpu.VMEM((CH,), jnp.int32),
        pltpu.VMEM((CH,), jnp.float32),
        pltpu.VMEM((NP,), jnp.float32),
        pltpu.VMEM((NP,), jnp.float32),
        pltpu.VMEM((NP,), jnp.float32),
    ],
)
def _sc_gate(dst_hbm, src_hbm, a_hbm, b_hbm, deg_hbm, e_hbm,
             di_v, si_v, ev_v, a_v, b_v, deg_v):
    w = _worker_id()
    pltpu.sync_copy(a_hbm, a_v)
    pltpu.sync_copy(b_hbm, b_v)
    pltpu.sync_copy(deg_hbm, deg_v)

    def body(j, carry):
        base = pl.multiple_of(w * PERW + j * CH, CH)
        pltpu.sync_copy(dst_hbm.at[pl.ds(base, CH)], di_v)
        pltpu.sync_copy(src_hbm.at[pl.ds(base, CH)], si_v)
        for r in range(CH // 16):
            d16 = di_v[pl.ds(r * 16, 16)]
            s16 = si_v[pl.ds(r * 16, 16)]
            av = plsc.load_gather(a_v, [d16])
            bv = plsc.load_gather(b_v, [s16])
            dd = plsc.load_gather(deg_v, [d16])
            ds_ = plsc.load_gather(deg_v, [s16])
            zz = av + bv
            t = 1.0 - 2.0 / (jnp.exp(2.0 * zz) + 1.0)
            ev_v[pl.ds(r * 16, 16)] = 0.3 + t * dd * ds_
        pltpu.sync_copy(ev_v, e_hbm.at[pl.ds(base, CH)])
        return carry

    lax.fori_loop(0, NCHUNK, body, 0)


@functools.partial(
    pl.kernel,
    out_type=jax.ShapeDtypeStruct((EP, H), jnp.float32),
    mesh=_mesh,
    compiler_params=pltpu.CompilerParams(use_tc_tiling_on_sc=False),
    scratch_types=[
        pltpu.VMEM((CH,), jnp.int32),
        pltpu.VMEM((CH, H), jnp.float32),
        pltpu.SemaphoreType.DMA,
    ],
)
def _sc_gather(table_hbm, src_hbm, out_hbm, idx_v, rows_v, sem):
    w = _worker_id()

    def body(j, carry):
        base = pl.multiple_of(w * PERW + j * CH, CH)
        pltpu.sync_copy(src_hbm.at[pl.ds(base, CH)], idx_v)
        pltpu.async_copy(table_hbm.at[idx_v], rows_v, sem).wait()
        pltpu.sync_copy(rows_v, out_hbm.at[pl.ds(base, CH), :])
        return carry

    lax.fori_loop(0, NCHUNK, body, 0)


@functools.partial(
    pl.kernel,
    out_type=jax.ShapeDtypeStruct((2, NP, H), jnp.float32),
    mesh=_mesh,
    compiler_params=pltpu.CompilerParams(use_tc_tiling_on_sc=False),
    scratch_types=[
        pltpu.VMEM((CH,), jnp.int32),
        pltpu.VMEM((CH, H), jnp.float32),
        pltpu.VMEM((CH, H), jnp.float32),
        pltpu.VMEM_SHARED((NP, H), jnp.float32),
    ],
)
def _sc_scatter(msg_hbm, dst_hbm, aggp_hbm, idx_v, msg_v, zero_v, agg_sp):
    c = lax.axis_index("c")
    s = lax.axis_index("s")
    w = _worker_id()
    z16 = jnp.zeros((16,), jnp.float32)

    def fill_zero(i, carry):
        zero_v[i, pl.ds(0, 16)] = z16
        zero_v[i, pl.ds(16, 16)] = z16
        return carry

    lax.fori_loop(0, CH, fill_zero, 0)
    for q in range(ROWS_PER_TILE // CH):
        pltpu.sync_copy(zero_v,
                        agg_sp.at[pl.ds(s * ROWS_PER_TILE + q * CH, CH), :])
    plsc.subcore_barrier()

    def body(j, carry):
        base = pl.multiple_of(w * PERW + j * CH, CH)
        pltpu.sync_copy(dst_hbm.at[pl.ds(base, CH)], idx_v)
        pltpu.sync_copy(msg_hbm.at[pl.ds(base, CH), :], msg_v)
        pltpu.sync_copy(msg_v, agg_sp.at[idx_v], add=True)
        return carry

    lax.fori_loop(0, NCHUNK, body, 0)
    plsc.subcore_barrier()
    pltpu.sync_copy(agg_sp.at[pl.ds(s * ROWS_PER_TILE, ROWS_PER_TILE), :],
                    aggp_hbm.at[c, pl.ds(s * ROWS_PER_TILE, ROWS_PER_TILE), :])


def _tc_project_body(x_ref, wpt_ref, bp_ref, gab_ref, gbh_ref,
                     h_ref, ab_ref):
    hv = jnp.maximum(
        jnp.dot(x_ref[...], wpt_ref[...],
                preferred_element_type=jnp.float32) + bp_ref[...], 0.0)
    h_ref[...] = hv
    ab_ref[...] = jnp.dot(hv, gab_ref[...],
                          preferred_element_type=jnp.float32) + gbh_ref[...]


def _tc_project(xp, wpt, bp_row, gab, gbh):
    return pl.pallas_call(
        _tc_project_body,
        out_shape=(jax.ShapeDtypeStruct((NP, H), jnp.float32),
                   jax.ShapeDtypeStruct((NP, 2), jnp.float32)),
    )(xp, wpt, bp_row, gab, gbh)


_BE = 2048


def _tc_msg_body(g_ref, ea_ref, e_ref, w2_ref, msg_ref):
    u = jnp.dot(g_ref[...], w2_ref[...], preferred_element_type=jnp.float32)
    acc = u[:, 544:576] + e_ref[...] * u[:, 512:544]
    for k in range(16):
        acc = acc + ea_ref[:, k:k + 1] * u[:, 32 * k:32 * k + 32]
    msg_ref[...] = acc


def _tc_msg(g, ea, ev, w2):
    return pl.pallas_call(
        _tc_msg_body,
        grid=(EP // _BE,),
        in_specs=[
            pl.BlockSpec((_BE, H), lambda i: (i, 0)),
            pl.BlockSpec((_BE, 16), lambda i: (i, 0)),
            pl.BlockSpec((_BE, 1), lambda i: (i, 0)),
            pl.BlockSpec((H, 576), lambda i: (0, 0)),
        ],
        out_specs=pl.BlockSpec((_BE, H), lambda i: (i, 0)),
        out_shape=jax.ShapeDtypeStruct((EP, H), jnp.float32),
    )(g, ea, ev, w2)


def _tc_gru_body(aggp_ref, hid_ref, bconv_ref, wih_ref, whh_ref,
                 bih_ref, bhh_ref, out_ref):
    agg = aggp_ref[0] + aggp_ref[1] + bconv_ref[...]
    xg = jnp.maximum(agg, 0.0)
    hid = hid_ref[...]
    gi = jnp.dot(xg, wih_ref[...],
                 preferred_element_type=jnp.float32) + bih_ref[...]
    gh = jnp.dot(hid, whh_ref[...],
                 preferred_element_type=jnp.float32) + bhh_ref[...]
    r = jax.nn.sigmoid(gi[:, :H] + gh[:, :H])
    z = jax.nn.sigmoid(gi[:, H:2 * H] + gh[:, H:2 * H])
    n = jnp.tanh(gi[:, 2 * H:] + r * gh[:, 2 * H:])
    out_ref[...] = (1.0 - z) * n + z * hid


def _tc_gru(aggp, hidden, bconv_row, wih_t, whh_t, bih_row, bhh_row):
    return pl.pallas_call(
        _tc_gru_body,
        out_shape=jax.ShapeDtypeStruct((NP, H), jnp.float32),
    )(aggp, hidden, bconv_row, wih_t, whh_t, bih_row, bhh_row)


def _tc_readout_body(h_ref, nf_ref, wih_ref, whh_ref, blstm_ref,
                     wst_ref, bs_ref, pa_ref, out_ref):
    na = jnp.concatenate([h_ref[:N, :], nf_ref[:N, :]], axis=1)
    hl = jnp.zeros((1, 64), jnp.float32)
    cl = jnp.zeros((1, 64), jnp.float32)
    qs = jnp.zeros((1, 128), jnp.float32)
    for _ in range(3):
        gates = (jnp.dot(qs, wih_ref[...], preferred_element_type=jnp.float32)
                 + jnp.dot(hl, whh_ref[...],
                           preferred_element_type=jnp.float32)
                 + blstm_ref[...])
        i = jax.nn.sigmoid(gates[:, :64])
        f = jax.nn.sigmoid(gates[:, 64:128])
        g = jnp.tanh(gates[:, 128:192])
        o = jax.nn.sigmoid(gates[:, 192:])
        cl = f * cl + i * g
        hl = o * jnp.tanh(cl)
        esc = jnp.sum(na * hl, axis=1, keepdims=True)
        m = jnp.max(esc)
        al = jnp.exp(esc - m)
        al = al / jnp.sum(al)
        ro = jnp.sum(al * na, axis=0, keepdims=True)
        qs = jnp.concatenate([hl, ro], axis=1)
    out = jnp.dot(qs, wst_ref[...],
                  preferred_element_type=jnp.float32) + bs_ref[...]
    pa = pa_ref[0, 0]
    out_ref[...] = jnp.where(out >= 0.0, out, pa * out)


def _tc_readout(h, nf, lwih_t, lwhh_t, blstm_row, ws_t, bs_row, pa):
    return pl.pallas_call(
        _tc_readout_body,
        out_shape=jax.ShapeDtypeStruct((1, 1024), jnp.float32),
    )(h, nf, lwih_t, lwhh_t, blstm_row, ws_t, bs_row, pa)


def kernel(x, edge_index, edge_attr, Wp, bp, gW, gb, Wen, ben, b_conv,
           gru_Wih, gru_Whh, gru_bih, gru_bhh, lstm_Wih, lstm_Whh,
           lstm_bih, lstm_bhh, Ws, bs, prelu_a):
    f32 = jnp.float32
    src = edge_index[0]
    dst = edge_index[1]
    pad = jnp.full((EP - E,), N, jnp.int32)
    src_p = jnp.concatenate([src, pad])
    dst_p = jnp.concatenate([dst, pad])
    ea_p = jnp.zeros((EP, 16), f32).at[:E].set(edge_attr)
    xp = jnp.zeros((NP, DIN), f32).at[:N].set(x)

    wpt = Wp.T
    bp_row = bp[None, :]
    gab = jnp.stack([gW[0, :H], gW[0, H:]], axis=1)
    gbh = jnp.full((1, 2), gb[0] * 0.5, f32)
    w2 = jnp.concatenate(
        [Wen.reshape(H, H, 17).transpose(0, 2, 1).reshape(H, 544),
         ben.reshape(H, H)], axis=1)
    wih_t = gru_Wih.T
    whh_t = gru_Whh.T
    bih_row = gru_bih[None, :]
    bhh_row = gru_bhh[None, :]
    bconv_row = b_conv[None, :]
    lwih_t = lstm_Wih.T
    lwhh_t = lstm_Whh.T
    blstm_row = (lstm_bih + lstm_bhh)[None, :]
    ws_t = Ws.T
    bs_row = bs[None, :]
    pa = jnp.reshape(prelu_a, (1, 1)).astype(f32)

    h, ab = _tc_project(xp, wpt, bp_row, gab, gbh)
    degp = _sc_degree(dst_p)
    deg = degp[0] + degp[1]
    ev = _sc_gate(dst_p, src_p, ab[:, 0], ab[:, 1], deg)
    ev2 = ev[:, None]

    hidden = h
    nf = h
    for _ in range(3):
        g = _sc_gather(nf, src_p)
        msg = _tc_msg(g, ea_p, ev2, w2)
        aggp = _sc_scatter(msg, dst_p)
        hidden = _tc_gru(aggp, hidden, bconv_row, wih_t, whh_t,
                         bih_row, bhh_row)
        nf = hidden

    return _tc_readout(h, nf, lwih_t, lwhh_t, blstm_row, ws_t, bs_row, pa)


# speedup vs baseline: 4.6249x; 2.3114x over previous
import functools

import jax
import jax.numpy as jnp
from jax import lax
from jax.experimental import pallas as pl
from jax.experimental.pallas import tpu as pltpu
from jax.experimental.pallas import tpu_sc as plsc

N = 10000
E = 160000
NP = 10240
EP = 163840
H = 32
DIN = 128
NW = 32
CH = 128
NCHUNK = EP // (NW * CH)
PERW = EP // NW
ROWS_PER_TILE = NP // 16

_mesh = plsc.VectorSubcoreMesh(core_axis_name="c", subcore_axis_name="s")


def _worker_id():
    return lax.axis_index("s") * 2 + lax.axis_index("c")


@functools.partial(
    pl.kernel,
    out_type=jax.ShapeDtypeStruct((2, NP), jnp.float32),
    mesh=_mesh,
    scratch_types=[
        pltpu.VMEM((NCHUNK, CH), jnp.int32),
        pltpu.VMEM((CH,), jnp.float32),
        pltpu.VMEM((ROWS_PER_TILE,), jnp.float32),
        pltpu.VMEM_SHARED((NP,), jnp.float32),
        [pltpu.SemaphoreType.DMA] * 4,
    ],
)
def _sc_degree(dst3_hbm, degp_hbm, idx_v, ones_v, zero_v, deg_sp, ssem):
    c = lax.axis_index("c")
    s = lax.axis_index("s")
    w = _worker_id()
    z16 = jnp.zeros((16,), jnp.float32)
    o16 = jnp.ones((16,), jnp.float32)

    def fill_zero(i, carry):
        zero_v[pl.ds(i * 16, 16)] = z16
        return carry

    lax.fori_loop(0, ROWS_PER_TILE // 16, fill_zero, 0)

    def fill_one(i, carry):
        ones_v[pl.ds(i * 16, 16)] = o16
        return carry

    lax.fori_loop(0, CH // 16, fill_one, 0)
    pltpu.sync_copy(zero_v, deg_sp.at[pl.ds(s * ROWS_PER_TILE, ROWS_PER_TILE)])
    pltpu.sync_copy(dst3_hbm.at[w], idx_v)
    plsc.subcore_barrier()

    sh = [None] * NCHUNK
    for j in range(NCHUNK):
        sh[j] = pltpu.async_copy(ones_v, deg_sp.at[idx_v.at[j]],
                                 ssem[j % 4], add=True)
        if j - 3 >= 0:
            sh[j - 3].wait()
    for j in range(NCHUNK - 3, NCHUNK):
        sh[j].wait()
    plsc.subcore_barrier()
    pltpu.sync_copy(deg_sp.at[pl.ds(s * ROWS_PER_TILE, ROWS_PER_TILE)],
                    degp_hbm.at[c, pl.ds(s * ROWS_PER_TILE, ROWS_PER_TILE)])


@functools.partial(
    pl.kernel,
    out_type=jax.ShapeDtypeStruct((EP,), jnp.float32),
    mesh=_mesh,
    compiler_params=pltpu.CompilerParams(needs_layout_passes=False),
    scratch_types=[
        pltpu.VMEM((PERW,), jnp.int32),
        pltpu.VMEM((PERW,), jnp.int32),
        pltpu.VMEM((PERW,), jnp.float32),
        pltpu.VMEM((NP,), jnp.float32),
        pltpu.VMEM((NP,), jnp.float32),
        pltpu.VMEM((NP,), jnp.float32),
    ],
)
def _sc_gate(dst_hbm, src_hbm, a_hbm, b_hbm, deg_hbm, e_hbm,
             di_v, si_v, ev_v, a_v, b_v, deg_v):
    w = _worker_id()
    base = pl.multiple_of(w * PERW, CH)
    pltpu.sync_copy(dst_hbm.at[pl.ds(base, PERW)], di_v)
    pltpu.sync_copy(src_hbm.at[pl.ds(base, PERW)], si_v)
    pltpu.sync_copy(a_hbm, a_v)
    pltpu.sync_copy(b_hbm, b_v)
    pltpu.sync_copy(deg_hbm, deg_v)

    def body(r, carry):
        d16 = di_v[pl.ds(r * 16, 16)]
        s16 = si_v[pl.ds(r * 16, 16)]
        av = plsc.load_gather(a_v, [d16])
        bv = plsc.load_gather(b_v, [s16])
        dd = plsc.load_gather(deg_v, [d16])
        ds_ = plsc.load_gather(deg_v, [s16])
        zz = av + bv
        t = 1.0 - 2.0 / (jnp.exp(2.0 * zz) + 1.0)
        ev_v[pl.ds(r * 16, 16)] = 0.3 + t * dd * ds_
        return carry

    lax.fori_loop(0, PERW // 16, body, 0)
    pltpu.sync_copy(ev_v, e_hbm.at[pl.ds(base, PERW)])


GCH = 512
NG = PERW // GCH


@functools.partial(
    pl.kernel,
    out_type=jax.ShapeDtypeStruct((EP, H), jnp.float32),
    mesh=_mesh,
    compiler_params=pltpu.CompilerParams(use_tc_tiling_on_sc=False),
    scratch_types=[
        pltpu.VMEM((PERW,), jnp.int32),
        pltpu.VMEM((2, GCH, H), jnp.float32),
        [pltpu.SemaphoreType.DMA, pltpu.SemaphoreType.DMA],
        [pltpu.SemaphoreType.DMA, pltpu.SemaphoreType.DMA],
    ],
)
def _sc_gather(table_hbm, src_hbm, out_hbm, idx_v, rows_v, gsem, wsem):
    w = _worker_id()
    base = pl.multiple_of(w * PERW, CH)
    pltpu.sync_copy(src_hbm.at[pl.ds(base, PERW)], idx_v)
    gh = [None, None]
    wh = [None, None]
    gh[0] = pltpu.async_copy(table_hbm.at[idx_v.at[pl.ds(0, GCH)]],
                             rows_v.at[0], gsem[0])
    for j in range(NG):
        p = j & 1
        gh[p].wait()
        if j + 1 < NG:
            q = p ^ 1
            if wh[q] is not None:
                wh[q].wait()
            gh[q] = pltpu.async_copy(
                table_hbm.at[idx_v.at[pl.ds((j + 1) * GCH, GCH)]],
                rows_v.at[q], gsem[q])
        wh[p] = pltpu.async_copy(
            rows_v.at[p],
            out_hbm.at[pl.ds(base + j * GCH, GCH), :], wsem[p])
    wh[0].wait()
    wh[1].wait()


@functools.partial(
    pl.kernel,
    out_type=jax.ShapeDtypeStruct((2, NP, H), jnp.float32),
    mesh=_mesh,
    compiler_params=pltpu.CompilerParams(use_tc_tiling_on_sc=False),
    scratch_types=[
        pltpu.VMEM((NCHUNK, CH), jnp.int32),
        pltpu.VMEM((4, CH, H), jnp.float32),
        pltpu.VMEM((CH, H), jnp.float32),
        pltpu.VMEM_SHARED((NP, H), jnp.float32),
        [pltpu.SemaphoreType.DMA] * 4,
        [pltpu.SemaphoreType.DMA] * 4,
    ],
)
def _sc_scatter(msg_hbm, dst3_hbm, aggp_hbm, idx_v, msg_v, zero_v, agg_sp,
                lsem, ssem):
    c = lax.axis_index("c")
    s = lax.axis_index("s")
    w = _worker_id()
    z16 = jnp.zeros((16,), jnp.float32)

    def fill_zero(i, carry):
        zero_v[i, pl.ds(0, 16)] = z16
        zero_v[i, pl.ds(16, 16)] = z16
        return carry

    lax.fori_loop(0, CH, fill_zero, 0)
    for q in range(ROWS_PER_TILE // CH):
        pltpu.sync_copy(zero_v,
                        agg_sp.at[pl.ds(s * ROWS_PER_TILE + q * CH, CH), :])
    pltpu.sync_copy(dst3_hbm.at[w], idx_v)
    plsc.subcore_barrier()

    base = pl.multiple_of(w * PERW, CH)
    lh = [None] * NCHUNK
    sh = [None] * NCHUNK
    for j in range(2):
        lh[j] = pltpu.async_copy(
            msg_hbm.at[pl.ds(base + j * CH, CH), :], msg_v.at[j % 4],
            lsem[j % 4])
    for j in range(NCHUNK):
        lh[j].wait()
        sh[j] = pltpu.async_copy(msg_v.at[j % 4], agg_sp.at[idx_v.at[j]],
                                 ssem[j % 4], add=True)
        if j + 2 < NCHUNK:
            if j - 2 >= 0:
                sh[j - 2].wait()
            lh[j + 2] = pltpu.async_copy(
                msg_hbm.at[pl.ds(base + (j + 2) * CH, CH), :],
                msg_v.at[(j + 2) % 4], lsem[(j + 2) % 4])
    for j in range(NCHUNK - 4, NCHUNK):
        sh[j].wait()
    plsc.subcore_barrier()
    pltpu.sync_copy(agg_sp.at[pl.ds(s * ROWS_PER_TILE, ROWS_PER_TILE), :],
                    aggp_hbm.at[c, pl.ds(s * ROWS_PER_TILE, ROWS_PER_TILE), :])


def _tc_project_body(x_ref, wpt_ref, bp_ref, gab_ref, gbh_ref,
                     h_ref, ab_ref):
    hv = jnp.maximum(
        jnp.dot(x_ref[...], wpt_ref[...],
                preferred_element_type=jnp.float32) + bp_ref[...], 0.0)
    h_ref[...] = hv
    ab_ref[...] = jnp.dot(hv, gab_ref[...],
                          preferred_element_type=jnp.float32) + gbh_ref[...]


def _tc_project(xp, wpt, bp_row, gab, gbh):
    return pl.pallas_call(
        _tc_project_body,
        out_shape=(jax.ShapeDtypeStruct((NP, H), jnp.float32),
                   jax.ShapeDtypeStruct((NP, 2), jnp.float32)),
    )(xp, wpt, bp_row, gab, gbh)


_BE = 2048


def _tc_msg_body(g_ref, efx_ref, w2p_ref, s32_ref, msg_ref):
    u = jnp.dot(g_ref[...], w2p_ref[...], preferred_element_type=jnp.float32)
    ew = jnp.dot(efx_ref[...], s32_ref[...],
                 preferred_element_type=jnp.float32)
    p = u * ew
    a = (p[:, 0:128] + p[:, 128:256] + p[:, 256:384] + p[:, 384:512]
         + p[:, 512:640])
    msg_ref[...] = a[:, 0:32] + a[:, 32:64] + a[:, 64:96] + a[:, 96:128]


def _tc_msg(g, efx32, w2p, s32):
    return pl.pallas_call(
        _tc_msg_body,
        grid=(EP // _BE,),
        in_specs=[
            pl.BlockSpec((_BE, H), lambda i: (i, 0)),
            pl.BlockSpec((_BE, H), lambda i: (i, 0)),
            pl.BlockSpec((H, 640), lambda i: (0, 0)),
            pl.BlockSpec((H, 640), lambda i: (0, 0)),
        ],
        out_specs=pl.BlockSpec((_BE, H), lambda i: (i, 0)),
        out_shape=jax.ShapeDtypeStruct((EP, H), jnp.float32),
    )(g, efx32, w2p, s32)


def _tc_gru_body(aggp_ref, hid_ref, bconv_ref, wih_ref, whh_ref,
                 bih_ref, bhh_ref, out_ref):
    agg = aggp_ref[0] + aggp_ref[1] + bconv_ref[...]
    xg = jnp.maximum(agg, 0.0)
    hid = hid_ref[...]
    gi = jnp.dot(xg, wih_ref[...],
                 preferred_element_type=jnp.float32) + bih_ref[...]
    gh = jnp.dot(hid, whh_ref[...],
                 preferred_element_type=jnp.float32) + bhh_ref[...]
    r = jax.nn.sigmoid(gi[:, :H] + gh[:, :H])
    z = jax.nn.sigmoid(gi[:, H:2 * H] + gh[:, H:2 * H])
    n = jnp.tanh(gi[:, 2 * H:] + r * gh[:, 2 * H:])
    out_ref[...] = (1.0 - z) * n + z * hid


def _tc_gru(aggp, hidden, bconv_row, wih_t, whh_t, bih_row, bhh_row):
    return pl.pallas_call(
        _tc_gru_body,
        out_shape=jax.ShapeDtypeStruct((NP, H), jnp.float32),
    )(aggp, hidden, bconv_row, wih_t, whh_t, bih_row, bhh_row)


def _tc_readout_body(h_ref, nf_ref, wih_ref, whh_ref, blstm_ref,
                     wst_ref, bs_ref, pa_ref, out_ref):
    na = jnp.concatenate([h_ref[:N, :], nf_ref[:N, :]], axis=1)
    hl = jnp.zeros((1, 64), jnp.float32)
    cl = jnp.zeros((1, 64), jnp.float32)
    qs = jnp.zeros((1, 128), jnp.float32)
    for _ in range(3):
        gates = (jnp.dot(qs, wih_ref[...], preferred_element_type=jnp.float32)
                 + jnp.dot(hl, whh_ref[...],
                           preferred_element_type=jnp.float32)
                 + blstm_ref[...])
        i = jax.nn.sigmoid(gates[:, :64])
        f = jax.nn.sigmoid(gates[:, 64:128])
        g = jnp.tanh(gates[:, 128:192])
        o = jax.nn.sigmoid(gates[:, 192:])
        cl = f * cl + i * g
        hl = o * jnp.tanh(cl)
        esc = jnp.sum(na * hl, axis=1, keepdims=True)
        m = jnp.max(esc)
        al = jnp.exp(esc - m)
        al = al / jnp.sum(al)
        ro = jnp.sum(al * na, axis=0, keepdims=True)
        qs = jnp.concatenate([hl, ro], axis=1)
    out = jnp.dot(qs, wst_ref[...],
                  preferred_element_type=jnp.float32) + bs_ref[...]
    pa = pa_ref[0, 0]
    out_ref[...] = jnp.where(out >= 0.0, out, pa * out)


def _tc_readout(h, nf, lwih_t, lwhh_t, blstm_row, ws_t, bs_row, pa):
    return pl.pallas_call(
        _tc_readout_body,
        out_shape=jax.ShapeDtypeStruct((1, 1024), jnp.float32),
    )(h, nf, lwih_t, lwhh_t, blstm_row, ws_t, bs_row, pa)


def kernel(x, edge_index, edge_attr, Wp, bp, gW, gb, Wen, ben, b_conv,
           gru_Wih, gru_Whh, gru_bih, gru_bhh, lstm_Wih, lstm_Whh,
           lstm_bih, lstm_bhh, Ws, bs, prelu_a):
    f32 = jnp.float32
    src = edge_index[0]
    dst = edge_index[1]
    pad = jnp.full((EP - E,), N, jnp.int32)
    src_p = jnp.concatenate([src, pad])
    dst_p = jnp.concatenate([dst, pad])
    ea_p = jnp.zeros((EP, 16), f32).at[:E].set(edge_attr)
    xp = jnp.zeros((NP, DIN), f32).at[:N].set(x)

    wpt = Wp.T
    bp_row = bp[None, :]
    gab = jnp.stack([gW[0, :H], gW[0, H:]], axis=1)
    gbh = jnp.full((1, 2), gb[0] * 0.5, f32)
    w2 = jnp.concatenate(
        [Wen.reshape(H, H, 17).transpose(0, 2, 1).reshape(H, 544),
         ben.reshape(H, H)], axis=1)
    wih_t = gru_Wih.T
    whh_t = gru_Whh.T
    bih_row = gru_bih[None, :]
    bhh_row = gru_bhh[None, :]
    bconv_row = b_conv[None, :]
    lwih_t = lstm_Wih.T
    lwhh_t = lstm_Whh.T
    blstm_row = (lstm_bih + lstm_bhh)[None, :]
    ws_t = Ws.T
    bs_row = bs[None, :]
    pa = jnp.reshape(prelu_a, (1, 1)).astype(f32)

    dst3 = dst_p.reshape(NW, NCHUNK, CH)
    w2p = jnp.zeros((H, 640), f32).at[:, :576].set(w2)
    s32 = jnp.kron(jnp.eye(32, 20, dtype=f32), jnp.ones((1, 32), f32))

    h, ab = _tc_project(xp, wpt, bp_row, gab, gbh)
    degp = _sc_degree(dst3)
    deg = degp[0] + degp[1]
    ev = _sc_gate(dst_p, src_p, ab[:, 0], ab[:, 1], deg)
    efx32 = jnp.concatenate(
        [ea_p, ev[:, None], jnp.ones((EP, 1), f32),
         jnp.zeros((EP, 14), f32)], axis=1)

    hidden = h
    nf = h
    for _ in range(3):
        g = _sc_gather(nf, src_p)
        msg = _tc_msg(g, efx32, w2p, s32)
        aggp = _sc_scatter(msg, dst3)
        hidden = _tc_gru(aggp, hidden, bconv_row, wih_t, whh_t,
                         bih_row, bhh_row)
        nf = hidden

    return _tc_readout(h, nf, lwih_t, lwhh_t, blstm_row, ws_t, bs_row, pa)


# speedup vs baseline: 4.7284x; 1.0224x over previous
import functools

import jax
import jax.numpy as jnp
from jax import lax
from jax.experimental import pallas as pl
from jax.experimental.pallas import tpu as pltpu
from jax.experimental.pallas import tpu_sc as plsc

N = 10000
E = 160000
NP = 10240
EP = 163840
H = 32
DIN = 128
NW = 32
CH = 128
NCHUNK = EP // (NW * CH)
PERW = EP // NW
ROWS_PER_TILE = NP // 16

_mesh = plsc.VectorSubcoreMesh(core_axis_name="c", subcore_axis_name="s")


def _worker_id():
    return lax.axis_index("s") * 2 + lax.axis_index("c")


NROW = EP // CH
RPTILE = NROW // 16


@functools.partial(
    pl.kernel,
    out_type=jax.ShapeDtypeStruct((EP,), jnp.float32),
    mesh=_mesh,
    compiler_params=pltpu.CompilerParams(needs_layout_passes=False),
    scratch_types=[
        pltpu.VMEM((RPTILE, CH), jnp.int32),
        pltpu.VMEM((PERW,), jnp.int32),
        pltpu.VMEM((PERW,), jnp.int32),
        pltpu.VMEM((PERW,), jnp.float32),
        pltpu.VMEM((NP,), jnp.float32),
        pltpu.VMEM((NP,), jnp.float32),
        pltpu.VMEM((NP,), jnp.float32),
        pltpu.VMEM((CH,), jnp.float32),
        pltpu.VMEM((ROWS_PER_TILE,), jnp.float32),
        pltpu.VMEM_SHARED((NP,), jnp.float32),
        [pltpu.SemaphoreType.DMA] * 4,
    ],
)
def _sc_gate(dst2_hbm, dst_hbm, src_hbm, a_hbm, b_hbm, e_hbm,
             dgi_v, di_v, si_v, ev_v, a_v, b_v, deg_v, ones_v, zero_v,
             deg_sp, ssem):
    s = lax.axis_index("s")
    w = _worker_id()
    z16 = jnp.zeros((16,), jnp.float32)
    o16 = jnp.ones((16,), jnp.float32)

    def fill_zero(i, carry):
        zero_v[pl.ds(i * 16, 16)] = z16
        return carry

    lax.fori_loop(0, ROWS_PER_TILE // 16, fill_zero, 0)

    def fill_one(i, carry):
        ones_v[pl.ds(i * 16, 16)] = o16
        return carry

    lax.fori_loop(0, CH // 16, fill_one, 0)
    pltpu.sync_copy(zero_v, deg_sp.at[pl.ds(s * ROWS_PER_TILE, ROWS_PER_TILE)])
    pltpu.sync_copy(dst2_hbm.at[pl.ds(s * RPTILE, RPTILE), :], dgi_v)
    base = pl.multiple_of(w * PERW, CH)
    pltpu.sync_copy(dst_hbm.at[pl.ds(base, PERW)], di_v)
    pltpu.sync_copy(src_hbm.at[pl.ds(base, PERW)], si_v)
    pltpu.sync_copy(a_hbm, a_v)
    pltpu.sync_copy(b_hbm, b_v)
    plsc.subcore_barrier()

    sh = [None] * RPTILE
    for j in range(RPTILE):
        sh[j] = pltpu.async_copy(ones_v, deg_sp.at[dgi_v.at[j]],
                                 ssem[j % 4], add=True)
        if j - 3 >= 0:
            sh[j - 3].wait()
    for j in range(RPTILE - 3, RPTILE):
        sh[j].wait()
    plsc.subcore_barrier()
    pltpu.sync_copy(deg_sp, deg_v)

    def body(r, carry):
        d16 = di_v[pl.ds(r * 16, 16)]
        s16 = si_v[pl.ds(r * 16, 16)]
        av = plsc.load_gather(a_v, [d16])
        bv = plsc.load_gather(b_v, [s16])
        dd = plsc.load_gather(deg_v, [d16])
        ds_ = plsc.load_gather(deg_v, [s16])
        zz = av + bv
        t = 1.0 - 2.0 / (jnp.exp(2.0 * zz) + 1.0)
        ev_v[pl.ds(r * 16, 16)] = 0.3 + t * dd * ds_
        return carry

    lax.fori_loop(0, PERW // 16, body, 0)
    pltpu.sync_copy(ev_v, e_hbm.at[pl.ds(base, PERW)])


GCH = 1024
NG = PERW // GCH


@functools.partial(
    pl.kernel,
    out_type=jax.ShapeDtypeStruct((EP, H), jnp.float32),
    mesh=_mesh,
    compiler_params=pltpu.CompilerParams(use_tc_tiling_on_sc=False),
    scratch_types=[
        pltpu.VMEM((PERW,), jnp.int32),
        pltpu.VMEM((2, GCH, H), jnp.float32),
        [pltpu.SemaphoreType.DMA, pltpu.SemaphoreType.DMA],
        [pltpu.SemaphoreType.DMA, pltpu.SemaphoreType.DMA],
    ],
)
def _sc_gather(table_hbm, src_hbm, out_hbm, idx_v, rows_v, gsem, wsem):
    w = _worker_id()
    base = pl.multiple_of(w * PERW, CH)
    pltpu.sync_copy(src_hbm.at[pl.ds(base, PERW)], idx_v)
    gh = [None, None]
    wh = [None, None]
    gh[0] = pltpu.async_copy(table_hbm.at[idx_v.at[pl.ds(0, GCH)]],
                             rows_v.at[0], gsem[0])
    for j in range(NG):
        p = j & 1
        gh[p].wait()
        if j + 1 < NG:
            q = p ^ 1
            if wh[q] is not None:
                wh[q].wait()
            gh[q] = pltpu.async_copy(
                table_hbm.at[idx_v.at[pl.ds((j + 1) * GCH, GCH)]],
                rows_v.at[q], gsem[q])
        wh[p] = pltpu.async_copy(
            rows_v.at[p],
            out_hbm.at[pl.ds(base + j * GCH, GCH), :], wsem[p])
    wh[0].wait()
    wh[1].wait()


@functools.partial(
    pl.kernel,
    out_type=jax.ShapeDtypeStruct((2, NP, H), jnp.float32),
    mesh=_mesh,
    compiler_params=pltpu.CompilerParams(use_tc_tiling_on_sc=False),
    scratch_types=[
        pltpu.VMEM((NCHUNK, CH), jnp.int32),
        pltpu.VMEM((8, CH, H), jnp.float32),
        pltpu.VMEM((CH, H), jnp.float32),
        pltpu.VMEM_SHARED((NP, H), jnp.float32),
        [pltpu.SemaphoreType.DMA] * 8,
        [pltpu.SemaphoreType.DMA] * 8,
    ],
)
def _sc_scatter(msg_hbm, dst3_hbm, aggp_hbm, idx_v, msg_v, zero_v, agg_sp,
                lsem, ssem):
    c = lax.axis_index("c")
    s = lax.axis_index("s")
    w = _worker_id()
    z16 = jnp.zeros((16,), jnp.float32)

    def fill_zero(i, carry):
        zero_v[i, pl.ds(0, 16)] = z16
        zero_v[i, pl.ds(16, 16)] = z16
        return carry

    lax.fori_loop(0, CH, fill_zero, 0)
    for q in range(ROWS_PER_TILE // CH):
        pltpu.sync_copy(zero_v,
                        agg_sp.at[pl.ds(s * ROWS_PER_TILE + q * CH, CH), :])
    pltpu.sync_copy(dst3_hbm.at[w], idx_v)
    plsc.subcore_barrier()

    base = pl.multiple_of(w * PERW, CH)
    lh = [None] * NCHUNK
    sh = [None] * NCHUNK
    for j in range(4):
        lh[j] = pltpu.async_copy(
            msg_hbm.at[pl.ds(base + j * CH, CH), :], msg_v.at[j % 8],
            lsem[j % 8])
    for j in range(NCHUNK):
        lh[j].wait()
        sh[j] = pltpu.async_copy(msg_v.at[j % 8], agg_sp.at[idx_v.at[j]],
                                 ssem[j % 8], add=True)
        if j + 4 < NCHUNK:
            if j - 4 >= 0:
                sh[j - 4].wait()
            lh[j + 4] = pltpu.async_copy(
                msg_hbm.at[pl.ds(base + (j + 4) * CH, CH), :],
                msg_v.at[(j + 4) % 8], lsem[(j + 4) % 8])
    for j in range(NCHUNK - 8, NCHUNK):
        sh[j].wait()
    plsc.subcore_barrier()
    pltpu.sync_copy(agg_sp.at[pl.ds(s * ROWS_PER_TILE, ROWS_PER_TILE), :],
                    aggp_hbm.at[c, pl.ds(s * ROWS_PER_TILE, ROWS_PER_TILE), :])


def _tc_project_body(x_ref, wpt_ref, bp_ref, gab_ref, gbh_ref,
                     h_ref, ab_ref):
    hv = jnp.maximum(
        jnp.dot(x_ref[...], wpt_ref[...],
                preferred_element_type=jnp.float32) + bp_ref[...], 0.0)
    h_ref[...] = hv
    ab_ref[...] = jnp.dot(hv, gab_ref[...],
                          preferred_element_type=jnp.float32) + gbh_ref[...]


def _tc_project(xp, wpt, bp_row, gab, gbh):
    return pl.pallas_call(
        _tc_project_body,
        out_shape=(jax.ShapeDtypeStruct((NP, H), jnp.float32),
                   jax.ShapeDtypeStruct((NP, 2), jnp.float32)),
    )(xp, wpt, bp_row, gab, gbh)


_BE = 2048


def _tc_msg_body(g_ref, efx_ref, w2p_ref, s32_ref, msg_ref):
    u = jnp.dot(g_ref[...], w2p_ref[...], preferred_element_type=jnp.float32)
    ew = jnp.dot(efx_ref[...], s32_ref[...],
                 preferred_element_type=jnp.float32)
    p = u * ew
    a = (p[:, 0:128] + p[:, 128:256] + p[:, 256:384] + p[:, 384:512]
         + p[:, 512:640])
    msg_ref[...] = a[:, 0:32] + a[:, 32:64] + a[:, 64:96] + a[:, 96:128]


def _tc_msg(g, efx32, w2p, s32):
    return pl.pallas_call(
        _tc_msg_body,
        grid=(EP // _BE,),
        in_specs=[
            pl.BlockSpec((_BE, H), lambda i: (i, 0)),
            pl.BlockSpec((_BE, H), lambda i: (i, 0)),
            pl.BlockSpec((H, 640), lambda i: (0, 0)),
            pl.BlockSpec((H, 640), lambda i: (0, 0)),
        ],
        out_specs=pl.BlockSpec((_BE, H), lambda i: (i, 0)),
        out_shape=jax.ShapeDtypeStruct((EP, H), jnp.float32),
    )(g, efx32, w2p, s32)


def _tc_gru_body(aggp_ref, hid_ref, bconv_ref, wih_ref, whh_ref,
                 bih_ref, bhh_ref, out_ref):
    agg = aggp_ref[0] + aggp_ref[1] + bconv_ref[...]
    xg = jnp.maximum(agg, 0.0)
    hid = hid_ref[...]
    gi = jnp.dot(xg, wih_ref[...],
                 preferred_element_type=jnp.float32) + bih_ref[...]
    gh = jnp.dot(hid, whh_ref[...],
                 preferred_element_type=jnp.float32) + bhh_ref[...]
    r = jax.nn.sigmoid(gi[:, :H] + gh[:, :H])
    z = jax.nn.sigmoid(gi[:, H:2 * H] + gh[:, H:2 * H])
    n = jnp.tanh(gi[:, 2 * H:] + r * gh[:, 2 * H:])
    out_ref[...] = (1.0 - z) * n + z * hid


def _tc_gru(aggp, hidden, bconv_row, wih_t, whh_t, bih_row, bhh_row):
    return pl.pallas_call(
        _tc_gru_body,
        out_shape=jax.ShapeDtypeStruct((NP, H), jnp.float32),
    )(aggp, hidden, bconv_row, wih_t, whh_t, bih_row, bhh_row)


def _tc_readout_body(h_ref, nf_ref, wih_ref, whh_ref, blstm_ref,
                     wst_ref, bs_ref, pa_ref, out_ref):
    na = jnp.concatenate([h_ref[:N, :], nf_ref[:N, :]], axis=1)
    hl = jnp.zeros((1, 64), jnp.float32)
    cl = jnp.zeros((1, 64), jnp.float32)
    qs = jnp.zeros((1, 128), jnp.float32)
    for _ in range(3):
        gates = (jnp.dot(qs, wih_ref[...], preferred_element_type=jnp.float32)
                 + jnp.dot(hl, whh_ref[...],
                           preferred_element_type=jnp.float32)
                 + blstm_ref[...])
        i = jax.nn.sigmoid(gates[:, :64])
        f = jax.nn.sigmoid(gates[:, 64:128])
        g = jnp.tanh(gates[:, 128:192])
        o = jax.nn.sigmoid(gates[:, 192:])
        cl = f * cl + i * g
        hl = o * jnp.tanh(cl)
        esc = jnp.sum(na * hl, axis=1, keepdims=True)
        m = jnp.max(esc)
        al = jnp.exp(esc - m)
        al = al / jnp.sum(al)
        ro = jnp.sum(al * na, axis=0, keepdims=True)
        qs = jnp.concatenate([hl, ro], axis=1)
    out = jnp.dot(qs, wst_ref[...],
                  preferred_element_type=jnp.float32) + bs_ref[...]
    pa = pa_ref[0, 0]
    out_ref[...] = jnp.where(out >= 0.0, out, pa * out)


def _tc_readout(h, nf, lwih_t, lwhh_t, blstm_row, ws_t, bs_row, pa):
    return pl.pallas_call(
        _tc_readout_body,
        out_shape=jax.ShapeDtypeStruct((1, 1024), jnp.float32),
    )(h, nf, lwih_t, lwhh_t, blstm_row, ws_t, bs_row, pa)


def kernel(x, edge_index, edge_attr, Wp, bp, gW, gb, Wen, ben, b_conv,
           gru_Wih, gru_Whh, gru_bih, gru_bhh, lstm_Wih, lstm_Whh,
           lstm_bih, lstm_bhh, Ws, bs, prelu_a):
    f32 = jnp.float32
    src = edge_index[0]
    dst = edge_index[1]
    pad = jnp.full((EP - E,), N, jnp.int32)
    src_p = jnp.concatenate([src, pad])
    dst_p = jnp.concatenate([dst, pad])
    ea_p = jnp.zeros((EP, 16), f32).at[:E].set(edge_attr)
    xp = jnp.zeros((NP, DIN), f32).at[:N].set(x)

    wpt = Wp.T
    bp_row = bp[None, :]
    gab = jnp.stack([gW[0, :H], gW[0, H:]], axis=1)
    gbh = jnp.full((1, 2), gb[0] * 0.5, f32)
    w2 = jnp.concatenate(
        [Wen.reshape(H, H, 17).transpose(0, 2, 1).reshape(H, 544),
         ben.reshape(H, H)], axis=1)
    wih_t = gru_Wih.T
    whh_t = gru_Whh.T
    bih_row = gru_bih[None, :]
    bhh_row = gru_bhh[None, :]
    bconv_row = b_conv[None, :]
    lwih_t = lstm_Wih.T
    lwhh_t = lstm_Whh.T
    blstm_row = (lstm_bih + lstm_bhh)[None, :]
    ws_t = Ws.T
    bs_row = bs[None, :]
    pa = jnp.reshape(prelu_a, (1, 1)).astype(f32)

    dst3 = dst_p.reshape(NW, NCHUNK, CH)
    dst2 = dst_p.reshape(NROW, CH)
    w2p = jnp.zeros((H, 640), f32).at[:, :576].set(w2)
    s32 = jnp.kron(jnp.eye(32, 20, dtype=f32), jnp.ones((1, 32), f32))

    h, ab = _tc_project(xp, wpt, bp_row, gab, gbh)
    ev = _sc_gate(dst2, dst_p, src_p, ab[:, 0], ab[:, 1])
    efx32 = jnp.concatenate(
        [ea_p, ev[:, None], jnp.ones((EP, 1), f32),
         jnp.zeros((EP, 14), f32)], axis=1)

    hidden = h
    nf = h
    for _ in range(3):
        g = _sc_gather(nf, src_p)
        msg = _tc_msg(g, efx32, w2p, s32)
        aggp = _sc_scatter(msg, dst3)
        hidden = _tc_gru(aggp, hidden, bconv_row, wih_t, whh_t,
                         bih_row, bhh_row)
        nf = hidden

    return _tc_readout(h, nf, lwih_t, lwhh_t, blstm_row, ws_t, bs_row, pa)
